# Initial kernel scaffold; baseline (speedup 1.0000x reference)
import jax
import jax.numpy as jnp
from jax.experimental import pallas as pl


def kernel(tokens, token_to_embed_map):
    raise NotImplementedError("write your pallas kernel here")


# speedup vs baseline: 1.5478x; 1.5478x over previous
import functools
import jax
import jax.numpy as jnp
from jax import lax
from jax.experimental import pallas as pl
from jax.experimental.pallas import tpu as pltpu
from jax.experimental.pallas import tpu_sc as plsc

D_VOCAB = 100000
D_MODEL = 768
BATCH = 4
SEQ_LEN = 8192

NC = 2
NS = 16
NW = NC * NS
B_TOTAL = BATCH * SEQ_LEN
B_PER_W = B_TOTAL // NW
CHUNK = 64
N_CHUNKS = B_PER_W // CHUNK


def _embed_body(tokens_hbm, table_hbm, out_hbm, idx_v, rows_v, sem):
    wid = lax.axis_index("s") * NC + lax.axis_index("c")
    base = wid * B_PER_W
    pltpu.sync_copy(tokens_hbm.at[pl.ds(base, B_PER_W)], idx_v)

    @pl.loop(0, N_CHUNKS)
    def _(g):
        off = g * CHUNK
        pltpu.async_copy(
            table_hbm.at[idx_v.at[pl.ds(off, CHUNK)]], rows_v, sem
        ).wait()
        pltpu.sync_copy(rows_v, out_hbm.at[pl.ds(base + off, CHUNK)])


@jax.jit
def _embed(tokens_flat, table):
    mesh = plsc.VectorSubcoreMesh(core_axis_name="c", subcore_axis_name="s")
    return pl.kernel(
        _embed_body,
        out_type=jax.ShapeDtypeStruct((B_TOTAL, D_MODEL), jnp.float32),
        mesh=mesh,
        scratch_types=[
            pltpu.VMEM((B_PER_W,), jnp.int32),
            pltpu.VMEM((CHUNK, D_MODEL), jnp.float32),
            pltpu.SemaphoreType.DMA,
        ],
    )(tokens_flat, table)


def kernel(tokens, token_to_embed_map):
    tokens_flat = tokens.reshape(-1).astype(jnp.int32)
    out = _embed(tokens_flat, token_to_embed_map)
    return out.reshape(BATCH, SEQ_LEN, D_MODEL)


# speedup vs baseline: 1.6648x; 1.0755x over previous
import functools
import jax
import jax.numpy as jnp
from jax import lax
from jax.experimental import pallas as pl
from jax.experimental.pallas import tpu as pltpu
from jax.experimental.pallas import tpu_sc as plsc

D_VOCAB = 100000
D_MODEL = 768
BATCH = 4
SEQ_LEN = 8192

NC = 2
NS = 16
NW = NC * NS
B_TOTAL = BATCH * SEQ_LEN
B_PER_W = B_TOTAL // NW
CHUNK = 64
N_CHUNKS = B_PER_W // CHUNK


def _embed_body(tokens_hbm, table_hbm, out_hbm,
                idx_v, rows0, rows1, gsem0, gsem1, wsem0, wsem1):
    wid = lax.axis_index("s") * NC + lax.axis_index("c")
    base = wid * B_PER_W
    pltpu.sync_copy(tokens_hbm.at[pl.ds(base, B_PER_W)], idx_v)

    bufs = (rows0, rows1)
    gsems = (gsem0, gsem1)
    wsems = (wsem0, wsem1)

    def gather(g):
        return pltpu.async_copy(
            table_hbm.at[idx_v.at[pl.ds(g * CHUNK, CHUNK)]],
            bufs[g % 2], gsems[g % 2])

    def write(g):
        return pltpu.async_copy(
            bufs[g % 2], out_hbm.at[pl.ds(base + g * CHUNK, CHUNK)],
            wsems[g % 2])

    gds = [None] * N_CHUNKS
    wds = [None] * N_CHUNKS
    gds[0] = gather(0)
    for g in range(N_CHUNKS):
        if g + 1 < N_CHUNKS:
            if g >= 1:
                wds[g - 1].wait()
            gds[g + 1] = gather(g + 1)
        gds[g].wait()
        wds[g] = write(g)
    wds[N_CHUNKS - 2].wait()
    wds[N_CHUNKS - 1].wait()


@jax.jit
def _embed(tokens_flat, table):
    mesh = plsc.VectorSubcoreMesh(core_axis_name="c", subcore_axis_name="s")
    return pl.kernel(
        _embed_body,
        out_type=jax.ShapeDtypeStruct((B_TOTAL, D_MODEL), jnp.float32),
        mesh=mesh,
        scratch_types=[
            pltpu.VMEM((B_PER_W,), jnp.int32),
            pltpu.VMEM((CHUNK, D_MODEL), jnp.float32),
            pltpu.VMEM((CHUNK, D_MODEL), jnp.float32),
            pltpu.SemaphoreType.DMA,
            pltpu.SemaphoreType.DMA,
            pltpu.SemaphoreType.DMA,
            pltpu.SemaphoreType.DMA,
        ],
    )(tokens_flat, table)


def kernel(tokens, token_to_embed_map):
    tokens_flat = tokens.reshape(-1).astype(jnp.int32)
    out = _embed(tokens_flat, token_to_embed_map)
    return out.reshape(BATCH, SEQ_LEN, D_MODEL)


# speedup vs baseline: 1.6750x; 1.0061x over previous
import jax
import jax.numpy as jnp
from jax import lax
from jax.experimental import pallas as pl
from jax.experimental.pallas import tpu as pltpu
from jax.experimental.pallas import tpu_sc as plsc

D_VOCAB = 100000
D_MODEL = 768
BATCH = 4
SEQ_LEN = 8192

NC = 2
NS = 16
NW = NC * NS
B_TOTAL = BATCH * SEQ_LEN
B_PER_W = B_TOTAL // NW
CHUNK = 32
N_CHUNKS = B_PER_W // CHUNK
N_BUF = 4


def _embed_body(tokens_hbm, table_hbm, out_hbm, idx_v, *rest):
    bufs = rest[:N_BUF]
    gsems = rest[N_BUF:2 * N_BUF]
    wsems = rest[2 * N_BUF:3 * N_BUF]

    wid = lax.axis_index("s") * NC + lax.axis_index("c")
    base = wid * B_PER_W
    pltpu.sync_copy(tokens_hbm.at[pl.ds(base, B_PER_W)], idx_v)

    def gather(g):
        return pltpu.async_copy(
            table_hbm.at[idx_v.at[pl.ds(g * CHUNK, CHUNK)]],
            bufs[g % N_BUF], gsems[g % N_BUF])

    def write(g):
        return pltpu.async_copy(
            bufs[g % N_BUF], out_hbm.at[pl.ds(base + g * CHUNK, CHUNK)],
            wsems[g % N_BUF])

    gds = [None] * N_CHUNKS
    wds = [None] * N_CHUNKS
    gds[0] = gather(0)
    gds[1] = gather(1)
    for g in range(N_CHUNKS):
        if g + 2 < N_CHUNKS:
            if g >= 2:
                wds[g - 2].wait()
            gds[g + 2] = gather(g + 2)
        gds[g].wait()
        wds[g] = write(g)
    for g in range(N_CHUNKS - 4, N_CHUNKS):
        wds[g].wait()


@jax.jit
def _embed(tokens_flat, table):
    mesh = plsc.VectorSubcoreMesh(core_axis_name="c", subcore_axis_name="s")
    return pl.kernel(
        _embed_body,
        out_type=jax.ShapeDtypeStruct((B_TOTAL, D_MODEL), jnp.float32),
        mesh=mesh,
        scratch_types=(
            [pltpu.VMEM((B_PER_W,), jnp.int32)]
            + [pltpu.VMEM((CHUNK, D_MODEL), jnp.float32)] * N_BUF
            + [pltpu.SemaphoreType.DMA] * (2 * N_BUF)
        ),
    )(tokens_flat, table)


def kernel(tokens, token_to_embed_map):
    tokens_flat = tokens.reshape(-1).astype(jnp.int32)
    out = _embed(tokens_flat, token_to_embed_map)
    return out.reshape(BATCH, SEQ_LEN, D_MODEL)
